# Initial kernel scaffold; baseline (speedup 1.0000x reference)
import jax
import jax.numpy as jnp
from jax.experimental import pallas as pl


def kernel(x, ability_emb, item_emb, species_emb, move_emb, W, b):
    raise NotImplementedError("write your pallas kernel here")


# speedup vs baseline: 4.2825x; 4.2825x over previous
import functools

import jax
import jax.numpy as jnp
from jax import lax
from jax.experimental import pallas as pl
from jax.experimental.pallas import tpu as pltpu
from jax.experimental.pallas import tpu_sc as plsc

NUM_GROUPS = 6
GROUP_SIZE = 16
OUT_DIM = 64
GSTRIDE = 6320
SEG_OFF = (0, 1504, 1808, 2312, 3312, 4312, 5312)
SEG_N = (1500, 300, 500, 1000, 1000, 1000, 1000)
BIAS_OFF = 6312
TABLE_ROWS = NUM_GROUPS * GSTRIDE

NC, NS, LANES = 2, 16, 16
NW = NC * NS
C = 16
SLOTS = 8
ROWS_PER_CHUNK = C * NUM_GROUPS * SLOTS
N_IDX_REFS = ROWS_PER_CHUNK // 128


def _proj_body(sp_ref, ab_ref, it_ref, mv_ref, w_ref, b_ref, out_ref):
    wg = w_ref[0]

    def proj(tab, c0, c1, r0, n):
        out_ref[pl.ds(r0, n), :] = lax.dot_general(
            tab, wg[:, c0:c1], (((1,), (1,)), ((), ())),
            preferred_element_type=jnp.float32,
            precision=lax.Precision.HIGHEST)

    proj(sp_ref[...], 0, 32, 0, 1504)
    proj(ab_ref[...], 32, 48, 1504, 304)
    proj(it_ref[...], 48, 64, 1808, 504)
    for j in range(4):
        proj(mv_ref[...], 64 + 16 * j, 80 + 16 * j, 2312 + 1000 * j, 1000)
    out_ref[pl.ds(BIAS_OFF, 8), :] = jnp.broadcast_to(b_ref[0, 0], (8, OUT_DIM))


def _build_flat_table(sp_p, ab_p, it_p, mv, w_pad, b):
    full = lambda shape: pl.BlockSpec(shape, lambda g: (0,) * len(shape))
    return pl.pallas_call(
        _proj_body,
        grid=(NUM_GROUPS,),
        in_specs=[
            full((1504, 32)), full((304, 16)), full((504, 16)), full((1000, 16)),
            pl.BlockSpec((1, 64, 144), lambda g: (g, 0, 0)),
            pl.BlockSpec((1, 1, 64), lambda g: (g, 0, 0)),
        ],
        out_specs=pl.BlockSpec((GSTRIDE, OUT_DIM), lambda g: (g, 0)),
        out_shape=jax.ShapeDtypeStruct((TABLE_ROWS, OUT_DIM), jnp.float32),
    )(sp_p, ab_p, it_p, mv, w_pad, b)


def _sc_encode(flat_table, xf, wn):
    import numpy as np
    consts = np.zeros((1 + NUM_GROUPS, LANES), np.int32)
    consts[0, :7] = [n - 1 for n in SEG_N]
    for g in range(NUM_GROUPS):
        consts[1 + g, :7] = [g * GSTRIDE + o for o in SEG_OFF]
        consts[1 + g, 7] = g * GSTRIDE + BIAS_OFF
    consts = jnp.asarray(consts)
    n_tokens = xf.shape[0]
    tpw = n_tokens // NW
    n_chunks = tpw // C

    mesh = plsc.VectorSubcoreMesh(core_axis_name="c", subcore_axis_name="s")

    @functools.partial(
        pl.kernel, mesh=mesh,
        compiler_params=pltpu.CompilerParams(
            needs_layout_passes=False, use_tc_tiling_on_sc=False),
        out_type=jax.ShapeDtypeStruct((n_tokens, NUM_GROUPS * OUT_DIM), jnp.float32),
        scratch_types=(
            [pltpu.VMEM((C, 96), jnp.float32)]
            + [pltpu.VMEM((128,), jnp.int32) for _ in range(N_IDX_REFS)]
            + [pltpu.VMEM((ROWS_PER_CHUNK, OUT_DIM), jnp.float32),
               pltpu.VMEM((C, NUM_GROUPS * OUT_DIM), jnp.float32),
               pltpu.VMEM((NUM_GROUPS, 9, OUT_DIM), jnp.float32),
               pltpu.VMEM((1 + NUM_GROUPS, LANES), jnp.int32),
               pltpu.SemaphoreType.DMA]),
    )
    def run(table_hbm, xf_hbm, wn_hbm, consts_hbm, out_hbm, x_v,
            i0, i1, i2, i3, i4, i5, rows_v, out_v, wn_v, cv_v, gsem):
        idx_refs = (i0, i1, i2, i3, i4, i5)
        wid = lax.axis_index("s") * NC + lax.axis_index("c")
        pltpu.sync_copy(wn_hbm, wn_v)
        pltpu.sync_copy(consts_hbm, cv_v)
        lane = lax.broadcasted_iota(jnp.int32, (LANES,), 0)
        segmax = cv_v[0]
        off_vecs = [cv_v[1 + g] for g in range(NUM_GROUPS)]
        slot_mask = lane < SLOTS

        def chunk_body(ci, carry):
            base = wid * tpw + ci * C
            pltpu.sync_copy(xf_hbm.at[pl.ds(base, C)], x_v)

            def idx_body(t, _):
                for g in range(NUM_GROUPS):
                    xg = x_v[t, pl.ds(g * GROUP_SIZE, GROUP_SIZE)]
                    iv = jnp.minimum(jnp.maximum(xg.astype(jnp.int32), 0), segmax)
                    plsc.store_scatter(idx_refs[g], [lane * LANES + t],
                                       iv + off_vecs[g], mask=slot_mask)
                return 0

            lax.fori_loop(0, C, idx_body, 0)

            cps = [pltpu.async_copy(table_hbm.at[idx_refs[k]],
                                    rows_v.at[pl.ds(k * 128, 128)], gsem)
                   for k in range(N_IDX_REFS)]
            for cp in cps:
                cp.wait()

            def tok_body(t, _):
                for g in range(NUM_GROUPS):
                    rbase = (g * SLOTS + 7) * LANES + t
                    acc = [rows_v[rbase, pl.ds(c * LANES, LANES)] for c in range(4)]
                    for s in range(7):
                        r = (g * SLOTS + s) * LANES + t
                        for c in range(4):
                            acc[c] = acc[c] + rows_v[r, pl.ds(c * LANES, LANES)]
                    xg = x_v[t, pl.ds(g * GROUP_SIZE, GROUP_SIZE)]
                    for k in range(9):
                        xv = xg[7 + k]
                        for c in range(4):
                            acc[c] = acc[c] + xv * wn_v[g, k, pl.ds(c * LANES, LANES)]
                    for c in range(4):
                        out_v[t, pl.ds(g * OUT_DIM + c * LANES, LANES)] = (
                            jnp.maximum(acc[c], 0.0))
                return 0

            lax.fori_loop(0, C, tok_body, 0)
            pltpu.sync_copy(out_v, out_hbm.at[pl.ds(base, C)])
            return 0

        lax.fori_loop(0, n_chunks, chunk_body, 0)

    return run(flat_table, xf, wn, consts)


def kernel(x, ability_emb, item_emb, species_emb, move_emb, W, b):
    B, S, F = x.shape
    xf = x.reshape(B * S, F)

    sp_p = jnp.pad(species_emb, ((0, 4), (0, 0)))
    ab_p = jnp.pad(ability_emb, ((0, 4), (0, 0)))
    it_p = jnp.pad(item_emb, ((0, 4), (0, 0)))
    w_pad = jnp.pad(W, ((0, 0), (0, 0), (0, 7)))
    wn = jnp.transpose(W[:, :, 128:137], (0, 2, 1))

    flat_table = _build_flat_table(sp_p, ab_p, it_p, move_emb, w_pad, b[:, None, :])
    out = _sc_encode(flat_table, xf, wn)
    return out.reshape(B, S, NUM_GROUPS * OUT_DIM)
